# baseline (speedup 1.0000x reference)
import functools

import jax
import jax.numpy as jnp
from jax import lax
from jax.experimental import pallas as pl
from jax.experimental.pallas import tpu as pltpu

_BITS_07 = 0x3F333333


def _ohem_body(score_ref, tgt_ref, out_ref, loss_s, pred_s, acc_s, *, nb, kth):
    i = pl.program_id(0)

    @pl.when(i == 0)
    def _init():
        acc_s[...] = jnp.zeros_like(acc_s)

    @pl.when(i < nb)
    def _dense():
        x = score_ref[...]
        bl, c = x.shape
        t = tgt_ref[0]
        eye = (lax.broadcasted_iota(jnp.int32, (c, c), 0)
               == lax.broadcasted_iota(jnp.int32, (c, c), 1)).astype(jnp.float32)
        xt_ = lax.dot_general(eye, x, (((1,), (1,)), ((), ())),
                              preferred_element_type=jnp.float32)
        cls = lax.broadcasted_iota(jnp.int32, (c, bl), 0)
        e = jnp.exp(xt_)
        s = jnp.sum(e, axis=0, keepdims=True)
        tx = jnp.sum(jnp.where(cls == t, xt_, 0.0), axis=0, keepdims=True)
        loss = jnp.log(s) - tx
        pred = jnp.exp(tx) / s
        loss_s[pl.ds(i, 1), :] = loss
        pred_s[pl.ds(i, 1), :] = pred
        keep = pred < 0.7
        acc_s[0:1, :] += jnp.where(keep, loss, 0.0)
        acc_s[1:2, :] += keep.astype(jnp.float32)

    @pl.when(i == nb)
    def _select():
        c07 = jnp.sum(acc_s[1:2, :])

        def _fast(_):
            return jnp.sum(acc_s[0:1, :]) / c07

        def _search(_):
            bits = lax.bitcast_convert_type(pred_s[...], jnp.int32)

            def bs_body(_, carry):
                lo, hi = carry
                mid = lax.div(lo + hi, 2)
                cnt = jnp.sum((bits <= mid).astype(jnp.int32))
                geq = cnt >= kth + 1
                return (jnp.where(geq, lo, mid + 1), jnp.where(geq, mid, hi))

            lo, _ = lax.fori_loop(0, 31, bs_body,
                                  (jnp.int32(0), jnp.int32(1 << 30)))
            thr = jnp.maximum(lo, _BITS_07)
            keep = bits < thr
            num = jnp.sum(jnp.where(keep, loss_s[...], 0.0))
            den = jnp.sum(keep.astype(jnp.float32))
            return num / den

        result = lax.cond(c07 > jnp.float32(kth), _fast, _search, None)
        out_ref[...] = result[None, None]


def kernel(score, target):
    n, c = score.shape
    bl = 2048
    nb = n // bl
    kth = min(int(0.7 * n), n - 1)
    tgt3 = target.reshape(nb, 1, bl)
    out = pl.pallas_call(
        functools.partial(_ohem_body, nb=nb, kth=kth),
        grid=(nb + 1,),
        in_specs=[
            pl.BlockSpec((bl, c), lambda i: (jnp.minimum(i, nb - 1), 0)),
            pl.BlockSpec((1, 1, bl), lambda i: (jnp.minimum(i, nb - 1), 0, 0)),
        ],
        out_specs=pl.BlockSpec((1, 1), lambda i: (0, 0)),
        out_shape=jax.ShapeDtypeStruct((1, 1), jnp.float32),
        scratch_shapes=[
            pltpu.VMEM((nb, bl), jnp.float32),
            pltpu.VMEM((nb, bl), jnp.float32),
            pltpu.VMEM((2, bl), jnp.float32),
        ],
    )(score, tgt3)
    return out[0, 0]
